# baseline (speedup 1.0000x reference)
import functools

import jax
import jax.numpy as jnp
from jax import lax
from jax.experimental import pallas as pl
from jax.experimental.pallas import tpu as pltpu
from jax.experimental.pallas import tpu_sc as plsc

_B = 4096
_D = 1024
_NC = 2
_NS = 16
_NW = _NC * _NS
_L = 16
_BPW = _B // _NW
_CH = 32
_NCHUNK = _BPW // _CH


@functools.partial(
    pl.kernel,
    out_type=jax.ShapeDtypeStruct((_NW, _L), jnp.float32),
    mesh=plsc.VectorSubcoreMesh(core_axis_name="c", subcore_axis_name="s"),
    scratch_types=[
        pltpu.VMEM((_BPW,), jnp.int32),
        pltpu.VMEM((_CH, _D), jnp.float32),
        pltpu.VMEM((_CH, _D), jnp.float32),
        pltpu.VMEM((_L,), jnp.float32),
        pltpu.SemaphoreType.DMA,
        pltpu.SemaphoreType.DMA,
    ],
)
def _center_loss_partials(x_hbm, y_hbm, tab_hbm, out_hbm,
                          idx_v, xb, cb, accv, semx, semc):
    wid = lax.axis_index("s") * _NC + lax.axis_index("c")
    base = wid * _BPW
    pltpu.sync_copy(y_hbm.at[pl.ds(base, _BPW)], idx_v)
    acc = jnp.zeros((_L,), jnp.float32)
    for ch in range(_NCHUNK):
        row0 = base + ch * _CH
        cpx = pltpu.async_copy(x_hbm.at[pl.ds(row0, _CH)], xb, semx)
        cpc = pltpu.async_copy(tab_hbm.at[idx_v.at[pl.ds(ch * _CH, _CH)]],
                               cb, semc)
        cpx.wait()
        cpc.wait()

        def body(r, a):
            for j in range(_D // _L):
                xv = xb[r, pl.ds(j * _L, _L)]
                cv = cb[r, pl.ds(j * _L, _L)]
                dv = xv - cv
                a = a + dv * dv
            return a

        acc = lax.fori_loop(0, _CH, body, acc)
    accv[...] = acc
    pltpu.sync_copy(accv, out_hbm.at[wid])


def kernel(output_features, y_truth, feature_centers):
    batch = y_truth.shape[0]
    x = output_features.reshape(batch, -1)
    partials = _center_loss_partials(
        x, y_truth.astype(jnp.int32), feature_centers)
    return (0.5 / batch) * jnp.sum(partials)
